# baseline (speedup 1.0000x reference)
import jax
import jax.numpy as jnp
from jax.experimental import pallas as pl
from jax.experimental.pallas import tpu as pltpu

_BR = 128
_NBUF = 8


def _body(x_hbm, pe_hbm, o_hbm, xb, pb, ob, rsx, rsp, ws):
    nblk = x_hbm.shape[0] // _BR

    def read(i):
        s = i % _NBUF
        pltpu.make_async_copy(x_hbm.at[pl.ds(i * _BR, _BR)], xb.at[s], rsx.at[s]).start()
        pltpu.make_async_copy(pe_hbm.at[pl.ds(i * _BR, _BR)], pb.at[s], rsp.at[s]).start()

    for i in range(_NBUF):
        read(i)
    for i in range(nblk):
        s = i % _NBUF
        pltpu.make_async_copy(x_hbm.at[pl.ds(i * _BR, _BR)], xb.at[s], rsx.at[s]).wait()
        pltpu.make_async_copy(pe_hbm.at[pl.ds(i * _BR, _BR)], pb.at[s], rsp.at[s]).wait()
        if i >= _NBUF:
            j = i - _NBUF
            pltpu.make_async_copy(ob.at[s], o_hbm.at[pl.ds(j * _BR, _BR)], ws.at[s]).wait()
        ob[s] = xb[s] + pb[s]
        pltpu.make_async_copy(ob.at[s], o_hbm.at[pl.ds(i * _BR, _BR)], ws.at[s]).start()
        if i + _NBUF < nblk:
            read(i + _NBUF)
    for i in range(nblk - _NBUF, nblk):
        s = i % _NBUF
        pltpu.make_async_copy(ob.at[s], o_hbm.at[pl.ds(i * _BR, _BR)], ws.at[s]).wait()


@jax.jit
def _pe_add(x, position_embeddings):
    seq_len, d_model = x.shape
    return pl.pallas_call(
        _body,
        in_specs=[
            pl.BlockSpec(memory_space=pltpu.MemorySpace.HBM),
            pl.BlockSpec(memory_space=pltpu.MemorySpace.HBM),
        ],
        out_specs=pl.BlockSpec(memory_space=pltpu.MemorySpace.HBM),
        out_shape=jax.ShapeDtypeStruct((seq_len, d_model), x.dtype),
        scratch_shapes=[
            pltpu.VMEM((_NBUF, _BR, d_model), jnp.float32),
            pltpu.VMEM((_NBUF, _BR, d_model), jnp.float32),
            pltpu.VMEM((_NBUF, _BR, d_model), jnp.float32),
            pltpu.SemaphoreType.DMA((_NBUF,)),
            pltpu.SemaphoreType.DMA((_NBUF,)),
            pltpu.SemaphoreType.DMA((_NBUF,)),
        ],
    )(x, position_embeddings)


def kernel(x, position_embeddings):
    seq_len = x.shape[0]
    return _pe_add(x, position_embeddings[:seq_len])


# speedup vs baseline: 1.0103x; 1.0103x over previous
import jax
import jax.numpy as jnp
from jax.experimental import pallas as pl
from jax.experimental.pallas import tpu as pltpu

_BR = 256
_NBUF = 4


def _body(x_hbm, pe_hbm, o_hbm, xb, pb, ob, rsx, rsp, ws):
    nblk = x_hbm.shape[0] // _BR

    def read(i):
        s = i % _NBUF
        pltpu.make_async_copy(x_hbm.at[pl.ds(i * _BR, _BR)], xb.at[s], rsx.at[s]).start()
        pltpu.make_async_copy(pe_hbm.at[pl.ds(i * _BR, _BR)], pb.at[s], rsp.at[s]).start()

    for i in range(_NBUF):
        read(i)
    for i in range(nblk):
        s = i % _NBUF
        pltpu.make_async_copy(x_hbm.at[pl.ds(i * _BR, _BR)], xb.at[s], rsx.at[s]).wait()
        pltpu.make_async_copy(pe_hbm.at[pl.ds(i * _BR, _BR)], pb.at[s], rsp.at[s]).wait()
        if i >= _NBUF:
            j = i - _NBUF
            pltpu.make_async_copy(ob.at[s], o_hbm.at[pl.ds(j * _BR, _BR)], ws.at[s]).wait()
        ob[s] = xb[s] + pb[s]
        pltpu.make_async_copy(ob.at[s], o_hbm.at[pl.ds(i * _BR, _BR)], ws.at[s]).start()
        if i + _NBUF < nblk:
            read(i + _NBUF)
    for i in range(nblk - _NBUF, nblk):
        s = i % _NBUF
        pltpu.make_async_copy(ob.at[s], o_hbm.at[pl.ds(i * _BR, _BR)], ws.at[s]).wait()


@jax.jit
def _pe_add(x, position_embeddings):
    seq_len, d_model = x.shape
    return pl.pallas_call(
        _body,
        in_specs=[
            pl.BlockSpec(memory_space=pltpu.MemorySpace.HBM),
            pl.BlockSpec(memory_space=pltpu.MemorySpace.HBM),
        ],
        out_specs=pl.BlockSpec(memory_space=pltpu.MemorySpace.HBM),
        out_shape=jax.ShapeDtypeStruct((seq_len, d_model), x.dtype),
        scratch_shapes=[
            pltpu.VMEM((_NBUF, _BR, d_model), jnp.float32),
            pltpu.VMEM((_NBUF, _BR, d_model), jnp.float32),
            pltpu.VMEM((_NBUF, _BR, d_model), jnp.float32),
            pltpu.SemaphoreType.DMA((_NBUF,)),
            pltpu.SemaphoreType.DMA((_NBUF,)),
            pltpu.SemaphoreType.DMA((_NBUF,)),
        ],
    )(x, position_embeddings)


def kernel(x, position_embeddings):
    seq_len = x.shape[0]
    return _pe_add(x, position_embeddings[:seq_len])
